# baseline (speedup 1.0000x reference)
import functools

import jax
import jax.numpy as jnp
from jax import lax
from jax.experimental import pallas as pl
from jax.experimental.pallas import tpu as pltpu
from jax.experimental.pallas import tpu_sc as plsc

CUTOFF = 10.0
N_NODES = 10000
N_EDGES = 320000
HIDDEN = 128
NUM_GAUSS = 50

NC = 2
NS = 16
CH = 40
NH = 2
EH = N_EDGES // NH
EPC = EH // NC
EPT = EPC // NS
CPT = EPT // CH
NPAD = 10240
RPT = NPAD // NS
EB = 6400
EBR = EB // 128
FBH = EH // EB
NB = 2000


def _ssp(x):
    return jnp.log(0.5 + 0.5 * jnp.exp(x))


def _h1_body(h_ref, w_ref, b_ref, o_ref):
    o_ref[...] = (
        jnp.dot(h_ref[...].astype(jnp.bfloat16), w_ref[...],
                preferred_element_type=jnp.float32)
        + b_ref[...]
    )


def _filter_body(ea_ref, ew_ref, wm1_ref, bm1_ref, wm2_ref, bm2_ref, o_ref):
    x = jnp.dot(ea_ref[...].astype(jnp.bfloat16), wm1_ref[...],
                preferred_element_type=jnp.float32)
    x = _ssp(x + bm1_ref[...])
    x = jnp.dot(x.astype(jnp.bfloat16), wm2_ref[...],
                preferred_element_type=jnp.float32)
    x = _ssp(x + bm2_ref[...])
    phi = ew_ref[...].reshape(EBR, 128) * (jnp.pi / (2.0 * CUTOFF))
    u = phi * phi
    cphi = 0.999999953464 + u * (
        -0.499999053455 + u * (0.0416635846769 + u * (
            -0.0013853704264 + u * 0.00002315393167)))
    cenv = cphi * cphi
    prod = x.reshape(EBR, 128, HIDDEN) * cenv[:, :, None]
    o_ref[...] = prod.reshape(EB, HIDDEN)


def _out_body(pa_ref, pb_ref, w1_ref, b1_ref, w2_ref, b2_ref, o_ref):
    agg = (pa_ref[0] + pa_ref[1] + pb_ref[0] + pb_ref[1]).astype(jnp.bfloat16)
    x = _ssp(
        jnp.dot(agg, w1_ref[...], preferred_element_type=jnp.float32)
        + b1_ref[...]
    )
    o_ref[...] = (
        jnp.dot(x.astype(jnp.bfloat16), w2_ref[...],
                preferred_element_type=jnp.float32)
        + b2_ref[...]
    )


def _rep(shape):
    return pl.BlockSpec(shape, lambda i: (0,) * len(shape))


_h1_call = pl.pallas_call(
    _h1_body,
    grid=(N_NODES // NB,),
    in_specs=[
        pl.BlockSpec((NB, HIDDEN), lambda i: (i, 0)),
        _rep((HIDDEN, HIDDEN)),
        _rep((1, HIDDEN)),
    ],
    out_specs=pl.BlockSpec((NB, HIDDEN), lambda i: (i, 0)),
    out_shape=jax.ShapeDtypeStruct((N_NODES, HIDDEN), jnp.float32),
)


def _make_filter(off):
    return pl.pallas_call(
        _filter_body,
        grid=(FBH,),
        in_specs=[
            pl.BlockSpec((EB, NUM_GAUSS), lambda i, o=off: (i + o, 0)),
            pl.BlockSpec((1, 1, EB), lambda i, o=off: (i + o, 0, 0)),
            _rep((NUM_GAUSS, HIDDEN)),
            _rep((1, HIDDEN)),
            _rep((HIDDEN, HIDDEN)),
            _rep((1, HIDDEN)),
        ],
        out_specs=pl.BlockSpec((EB, HIDDEN), lambda i: (i, 0)),
        out_shape=jax.ShapeDtypeStruct((EH, HIDDEN), jnp.float32),
    )


_filter_a = _make_filter(0)
_filter_b = _make_filter(FBH)

_out_call = pl.pallas_call(
    _out_body,
    grid=(N_NODES // NB,),
    in_specs=[
        pl.BlockSpec((NC, NB, HIDDEN), lambda i: (0, i, 0)),
        pl.BlockSpec((NC, NB, HIDDEN), lambda i: (0, i, 0)),
        _rep((HIDDEN, HIDDEN)),
        _rep((1, HIDDEN)),
        _rep((HIDDEN, HIDDEN)),
        _rep((1, HIDDEN)),
    ],
    out_specs=pl.BlockSpec((NB, HIDDEN), lambda i: (i, 0)),
    out_shape=jax.ShapeDtypeStruct((N_NODES, HIDDEN), jnp.float32),
)


def _make_sc_body(eoff):
    def _sc_body(h1_hbm, wf_hbm, src_hbm, dst_hbm, zeros_hbm, out_hbm,
                 src_all,
                 dst_idx0, dst_idx1, rows0, rows1, wf0, wf1, agg,
                 gsem0, gsem1, wsem0, wsem1, dsem0, dsem1, ssem0, ssem1):
        c = lax.axis_index("c")
        s = lax.axis_index("s")
        lb = c * EPC + s * EPT
        tb = eoff + lb

        pltpu.sync_copy(zeros_hbm, agg.at[pl.ds(s * RPT, RPT)])
        pltpu.sync_copy(src_hbm.at[pl.ds(tb, EPT)], src_all)
        plsc.subcore_barrier()

        bufs = ((dst_idx0, rows0, wf0, gsem0, wsem0, dsem0, ssem0),
                (dst_idx1, rows1, wf1, gsem1, wsem1, dsem1, ssem1))

        def issue(b, k, first=False):
            dstb, gb, wb, gsem, wsem, dsem, ssem = bufs[b]

            @pl.when(k < CPT)
            def _():
                if not first:
                    @pl.when(k >= 2)
                    def _():
                        pltpu.make_async_copy(gb, agg.at[dstb], ssem).wait()
                pltpu.async_copy(
                    dst_hbm.at[pl.ds(tb + k * CH, CH)], dstb, dsem)
                pltpu.async_copy(
                    h1_hbm.at[src_all.at[pl.ds(k * CH, CH)]], gb, gsem)
                pltpu.async_copy(
                    wf_hbm.at[pl.ds(lb + k * CH, CH)], wb, wsem)

        def finish(b, k):
            dstb, gb, wb, gsem, wsem, dsem, ssem = bufs[b]

            @pl.when(k < CPT)
            def _():
                pltpu.make_async_copy(
                    h1_hbm.at[src_all.at[pl.ds(k * CH, CH)]], gb, gsem).wait()
                pltpu.make_async_copy(
                    wf_hbm.at[pl.ds(lb + k * CH, CH)], wb, wsem).wait()

                def mul_row(i, cc):
                    for j in range(HIDDEN // 16):
                        sl = pl.ds(j * 16, 16)
                        gb[i, sl] = gb[i, sl] * wb[i, sl]
                    return cc

                lax.fori_loop(0, CH, mul_row, 0)
                pltpu.make_async_copy(
                    dst_hbm.at[pl.ds(tb + k * CH, CH)], dstb, dsem).wait()
                pltpu.async_copy(gb, agg.at[dstb], ssem, add=True)

        issue(0, 0, first=True)

        def body(j, carry):
            k0 = 2 * j
            k1 = k0 + 1
            issue(1, k1)
            finish(0, k0)
            issue(0, k0 + 2)
            finish(1, k1)
            return carry

        lax.fori_loop(0, (CPT + 1) // 2, body, 0)

        pltpu.make_async_copy(rows0, agg.at[dst_idx0], ssem0).wait()
        pltpu.make_async_copy(rows1, agg.at[dst_idx1], ssem1).wait()
        plsc.subcore_barrier()
        pltpu.sync_copy(
            agg.at[pl.ds(s * RPT, RPT)],
            out_hbm.at[pl.ds(c * NPAD + s * RPT, RPT)],
        )

    return _sc_body


def _make_sc(eoff):
    return functools.partial(
        pl.kernel,
        out_type=jax.ShapeDtypeStruct((NC * NPAD, HIDDEN), jnp.float32),
        mesh=plsc.VectorSubcoreMesh(core_axis_name="c", subcore_axis_name="s"),
        scratch_types=[
            pltpu.VMEM((EPT,), jnp.int32),
            pltpu.VMEM((CH,), jnp.int32),
            pltpu.VMEM((CH,), jnp.int32),
            pltpu.VMEM((CH, HIDDEN), jnp.float32),
            pltpu.VMEM((CH, HIDDEN), jnp.float32),
            pltpu.VMEM((CH, HIDDEN), jnp.float32),
            pltpu.VMEM((CH, HIDDEN), jnp.float32),
            pltpu.VMEM_SHARED((NPAD, HIDDEN), jnp.float32),
            pltpu.SemaphoreType.DMA,
            pltpu.SemaphoreType.DMA,
            pltpu.SemaphoreType.DMA,
            pltpu.SemaphoreType.DMA,
            pltpu.SemaphoreType.DMA,
            pltpu.SemaphoreType.DMA,
            pltpu.SemaphoreType.DMA,
            pltpu.SemaphoreType.DMA,
        ],
    )(_make_sc_body(eoff))


_sc_a = _make_sc(0)
_sc_b = _make_sc(EH)


def kernel(h, edge_index, edge_weight, edge_attr,
           W_aw, b_aw, W_m1, b_m1, W_m2, b_m2, W_o1, b_o1, W_o2, b_o2):
    ei = edge_index.astype(jnp.int32)
    src, dst = ei[0], ei[1]
    bf16 = jnp.bfloat16
    ew3 = edge_weight.reshape(N_EDGES // EB, 1, EB)
    zeros = jnp.zeros((RPT, HIDDEN), jnp.float32)
    h1 = _h1_call(h, W_aw.astype(bf16), b_aw.reshape(1, HIDDEN))
    wf_a = _filter_a(edge_attr, ew3, W_m1.astype(bf16),
                     b_m1.reshape(1, HIDDEN), W_m2.astype(bf16),
                     b_m2.reshape(1, HIDDEN))
    pa = _sc_a(h1, wf_a, src, dst, zeros)
    wf_b = _filter_b(edge_attr, ew3, W_m1.astype(bf16),
                     b_m1.reshape(1, HIDDEN), W_m2.astype(bf16),
                     b_m2.reshape(1, HIDDEN))
    pb = _sc_b(h1, wf_b, src, dst, zeros)
    out = _out_call(pa.reshape(NC, NPAD, HIDDEN), pb.reshape(NC, NPAD, HIDDEN),
                    W_o1.astype(bf16), b_o1.reshape(1, HIDDEN),
                    W_o2.astype(bf16), b_o2.reshape(1, HIDDEN))
    return out


# speedup vs baseline: 1.0846x; 1.0846x over previous
import functools

import jax
import jax.numpy as jnp
from jax import lax
from jax.experimental import pallas as pl
from jax.experimental.pallas import tpu as pltpu
from jax.experimental.pallas import tpu_sc as plsc

CUTOFF = 10.0
N_NODES = 10000
N_EDGES = 320000
HIDDEN = 128
NUM_GAUSS = 50

NC = 2
NS = 16
CH = 40
NH = 2
EH = N_EDGES // NH
EPC = EH // NC
EPT = EPC // NS
CPT = EPT // CH
NPAD = 10240
RPT = NPAD // NS
EB = 6400
EBR = EB // 128
FBH = EH // EB
NB = 2000


import numpy as np

_HMASK = -65536
_P = np.arange(32)
_QPERM = np.concatenate(
    [32 * g + (_P >> 1) + 16 * (_P & 1) for g in range(HIDDEN // 32)])


def _ssp(x):
    return jnp.log(0.5 + 0.5 * jnp.exp(x))


def _h1_body(h_ref, w_ref, b_ref, o_ref):
    o_ref[...] = (
        jnp.dot(h_ref[...].astype(jnp.bfloat16), w_ref[...],
                preferred_element_type=jnp.float32)
        + b_ref[...]
    )


def _filter_body(ea_ref, ew_ref, wm1_ref, bm1_ref, wm2_ref, bm2_ref, o_ref):
    x = jnp.dot(ea_ref[...].astype(jnp.bfloat16), wm1_ref[...],
                preferred_element_type=jnp.float32)
    x = _ssp(x + bm1_ref[...])
    x = jnp.dot(x.astype(jnp.bfloat16), wm2_ref[...],
                preferred_element_type=jnp.float32)
    x = _ssp(x + bm2_ref[...])
    phi = ew_ref[...].reshape(EBR, 128) * (jnp.pi / (2.0 * CUTOFF))
    u = phi * phi
    cphi = 0.999999953464 + u * (
        -0.499999053455 + u * (0.0416635846769 + u * (
            -0.0013853704264 + u * 0.00002315393167)))
    cenv = cphi * cphi
    prod = x.reshape(EBR, 128, HIDDEN) * cenv[:, :, None]
    o_ref[...] = prod.reshape(EB, HIDDEN).astype(jnp.bfloat16)


def _out_body(pa_ref, pb_ref, w1_ref, b1_ref, w2_ref, b2_ref, o_ref):
    agg = (pa_ref[0] + pa_ref[1] + pb_ref[0] + pb_ref[1]).astype(jnp.bfloat16)
    x = _ssp(
        jnp.dot(agg, w1_ref[...], preferred_element_type=jnp.float32)
        + b1_ref[...]
    )
    o_ref[...] = (
        jnp.dot(x.astype(jnp.bfloat16), w2_ref[...],
                preferred_element_type=jnp.float32)
        + b2_ref[...]
    )


def _rep(shape):
    return pl.BlockSpec(shape, lambda i: (0,) * len(shape))


_h1_call = pl.pallas_call(
    _h1_body,
    grid=(N_NODES // NB,),
    in_specs=[
        pl.BlockSpec((NB, HIDDEN), lambda i: (i, 0)),
        _rep((HIDDEN, HIDDEN)),
        _rep((1, HIDDEN)),
    ],
    out_specs=pl.BlockSpec((NB, HIDDEN), lambda i: (i, 0)),
    out_shape=jax.ShapeDtypeStruct((N_NODES, HIDDEN), jnp.float32),
)


def _make_filter(off):
    return pl.pallas_call(
        _filter_body,
        grid=(FBH,),
        in_specs=[
            pl.BlockSpec((EB, NUM_GAUSS), lambda i, o=off: (i + o, 0)),
            pl.BlockSpec((1, 1, EB), lambda i, o=off: (i + o, 0, 0)),
            _rep((NUM_GAUSS, HIDDEN)),
            _rep((1, HIDDEN)),
            _rep((HIDDEN, HIDDEN)),
            _rep((1, HIDDEN)),
        ],
        out_specs=pl.BlockSpec((EB, HIDDEN), lambda i: (i, 0)),
        out_shape=jax.ShapeDtypeStruct((EH, HIDDEN), jnp.bfloat16),
    )


_filter_a = _make_filter(0)
_filter_b = _make_filter(FBH)

_out_call = pl.pallas_call(
    _out_body,
    grid=(N_NODES // NB,),
    in_specs=[
        pl.BlockSpec((NC, NB, HIDDEN), lambda i: (0, i, 0)),
        pl.BlockSpec((NC, NB, HIDDEN), lambda i: (0, i, 0)),
        _rep((HIDDEN, HIDDEN)),
        _rep((1, HIDDEN)),
        _rep((HIDDEN, HIDDEN)),
        _rep((1, HIDDEN)),
    ],
    out_specs=pl.BlockSpec((NB, HIDDEN), lambda i: (i, 0)),
    out_shape=jax.ShapeDtypeStruct((N_NODES, HIDDEN), jnp.float32),
)


def _make_sc_body(eoff):
    def _sc_body(h1_hbm, wf_hbm, src_hbm, dst_hbm, zeros_hbm, out_hbm,
                 src_all,
                 dst_idx0, dst_idx1, rows0, rows1, wf0, wf1, agg,
                 gsem0, gsem1, wsem0, wsem1, dsem0, dsem1, ssem0, ssem1):
        c = lax.axis_index("c")
        s = lax.axis_index("s")
        lb = c * EPC + s * EPT
        tb = eoff + lb

        pltpu.sync_copy(zeros_hbm, agg.at[pl.ds(s * RPT, RPT)])
        pltpu.sync_copy(src_hbm.at[pl.ds(tb, EPT)], src_all)
        plsc.subcore_barrier()

        bufs = ((dst_idx0, rows0, wf0, gsem0, wsem0, dsem0, ssem0),
                (dst_idx1, rows1, wf1, gsem1, wsem1, dsem1, ssem1))

        def issue(b, k, first=False):
            dstb, gb, wb, gsem, wsem, dsem, ssem = bufs[b]

            @pl.when(k < CPT)
            def _():
                if not first:
                    @pl.when(k >= 2)
                    def _():
                        pltpu.make_async_copy(gb, agg.at[dstb], ssem).wait()
                pltpu.async_copy(
                    dst_hbm.at[pl.ds(tb + k * CH, CH)], dstb, dsem)
                pltpu.async_copy(
                    h1_hbm.at[src_all.at[pl.ds(k * CH, CH)]], gb, gsem)
                pltpu.async_copy(
                    wf_hbm.at[pl.ds(lb + k * CH, CH)], wb, wsem)

        def finish(b, k):
            dstb, gb, wb, gsem, wsem, dsem, ssem = bufs[b]

            @pl.when(k < CPT)
            def _():
                pltpu.make_async_copy(
                    h1_hbm.at[src_all.at[pl.ds(k * CH, CH)]], gb, gsem).wait()
                pltpu.make_async_copy(
                    wf_hbm.at[pl.ds(lb + k * CH, CH)], wb, wsem).wait()

                wbi = wb.bitcast(jnp.int32)

                def mul_row(i, cc):
                    for g in range(HIDDEN // 32):
                        ww = wbi[i, pl.ds(g * 16, 16)]
                        wlo = lax.bitcast_convert_type(ww << 16, jnp.float32)
                        whi = lax.bitcast_convert_type(ww & _HMASK,
                                                       jnp.float32)
                        slo = pl.ds(g * 32, 16)
                        shi = pl.ds(g * 32 + 16, 16)
                        gb[i, slo] = gb[i, slo] * wlo
                        gb[i, shi] = gb[i, shi] * whi
                    return cc

                lax.fori_loop(0, CH, mul_row, 0)
                pltpu.make_async_copy(
                    dst_hbm.at[pl.ds(tb + k * CH, CH)], dstb, dsem).wait()
                pltpu.async_copy(gb, agg.at[dstb], ssem, add=True)

        issue(0, 0, first=True)

        def body(j, carry):
            k0 = 2 * j
            k1 = k0 + 1
            issue(1, k1)
            finish(0, k0)
            issue(0, k0 + 2)
            finish(1, k1)
            return carry

        lax.fori_loop(0, (CPT + 1) // 2, body, 0)

        pltpu.make_async_copy(rows0, agg.at[dst_idx0], ssem0).wait()
        pltpu.make_async_copy(rows1, agg.at[dst_idx1], ssem1).wait()
        plsc.subcore_barrier()
        pltpu.sync_copy(
            agg.at[pl.ds(s * RPT, RPT)],
            out_hbm.at[pl.ds(c * NPAD + s * RPT, RPT)],
        )

    return _sc_body


def _make_sc(eoff):
    return functools.partial(
        pl.kernel,
        out_type=jax.ShapeDtypeStruct((NC * NPAD, HIDDEN), jnp.float32),
        mesh=plsc.VectorSubcoreMesh(core_axis_name="c", subcore_axis_name="s"),
        scratch_types=[
            pltpu.VMEM((EPT,), jnp.int32),
            pltpu.VMEM((CH,), jnp.int32),
            pltpu.VMEM((CH,), jnp.int32),
            pltpu.VMEM((CH, HIDDEN), jnp.float32),
            pltpu.VMEM((CH, HIDDEN), jnp.float32),
            pltpu.VMEM((CH, HIDDEN), jnp.bfloat16),
            pltpu.VMEM((CH, HIDDEN), jnp.bfloat16),
            pltpu.VMEM_SHARED((NPAD, HIDDEN), jnp.float32),
            pltpu.SemaphoreType.DMA,
            pltpu.SemaphoreType.DMA,
            pltpu.SemaphoreType.DMA,
            pltpu.SemaphoreType.DMA,
            pltpu.SemaphoreType.DMA,
            pltpu.SemaphoreType.DMA,
            pltpu.SemaphoreType.DMA,
            pltpu.SemaphoreType.DMA,
        ],
    )(_make_sc_body(eoff))


_sc_a = _make_sc(0)
_sc_b = _make_sc(EH)


def kernel(h, edge_index, edge_weight, edge_attr,
           W_aw, b_aw, W_m1, b_m1, W_m2, b_m2, W_o1, b_o1, W_o2, b_o2):
    ei = edge_index.astype(jnp.int32)
    src, dst = ei[0], ei[1]
    bf16 = jnp.bfloat16
    ew3 = edge_weight.reshape(N_EDGES // EB, 1, EB)
    zeros = jnp.zeros((RPT, HIDDEN), jnp.float32)
    h1 = _h1_call(h, W_aw.astype(bf16), b_aw.reshape(1, HIDDEN))
    wm2q = W_m2[:, _QPERM].astype(bf16)
    bm2q = b_m2[_QPERM].reshape(1, HIDDEN)
    wf_a = _filter_a(edge_attr, ew3, W_m1.astype(bf16),
                     b_m1.reshape(1, HIDDEN), wm2q, bm2q)
    pa = _sc_a(h1, wf_a, src, dst, zeros)
    wf_b = _filter_b(edge_attr, ew3, W_m1.astype(bf16),
                     b_m1.reshape(1, HIDDEN), wm2q, bm2q)
    pb = _sc_b(h1, wf_b, src, dst, zeros)
    out = _out_call(pa.reshape(NC, NPAD, HIDDEN), pb.reshape(NC, NPAD, HIDDEN),
                    W_o1.astype(bf16), b_o1.reshape(1, HIDDEN),
                    W_o2.astype(bf16), b_o2.reshape(1, HIDDEN))
    return out
